# baseline (speedup 1.0000x reference)
import jax
import jax.numpy as jnp
from jax import lax
from jax.experimental import pallas as pl
from jax.experimental.pallas import tpu as pltpu
from jax.experimental.pallas import tpu_sc as plsc

NC = 2
NS = 16
LANE = 16
CHUNK = 64


def _round_up(x, m):
    return (x + m - 1) // m * m


def _sc_mesh():
    return plsc.VectorSubcoreMesh(
        core_axis_name="c", subcore_axis_name="s", num_cores=NC,
        num_subcores=NS)


def _deg_remap_call(src3, dst3, n_deg, trash):
    n_chunks = src3.shape[1]
    nz = n_deg // NS

    def body(src_hbm, dst_hbm, deg_hbm, rmp_hbm,
             src_v, dst_v, val_v, rmp_v, buf_v, dsem, deg_sp):
        c = lax.axis_index("c")
        s = lax.axis_index("s")
        t = c * NS + s
        pltpu.sync_copy(src_hbm.at[t], src_v)
        pltpu.sync_copy(dst_hbm.at[t], dst_v)

        def zstep(i, _):
            buf_v[pl.ds(i * LANE, LANE)] = jnp.zeros((LANE,), jnp.float32)
            return 0
        lax.fori_loop(0, nz // LANE, zstep, 0)
        pltpu.sync_copy(buf_v, deg_sp.at[pl.ds(s * nz, nz)])
        plsc.subcore_barrier()

        def cstep(j, _):
            for u in range(CHUNK // LANE):
                sj = src_v[j, pl.ds(u * LANE, LANE)]
                dj = dst_v[j, pl.ds(u * LANE, LANE)]
                m = sj != dj
                val_v[j, pl.ds(u * LANE, LANE)] = jnp.where(m, 1.0, 0.0)
                rmp_v[j, pl.ds(u * LANE, LANE)] = jnp.where(m, dj, trash)
            return 0
        lax.fori_loop(0, n_chunks, cstep, 0)

        def fire(j, _):
            pltpu.async_copy(val_v.at[j], deg_sp.at[src_v.at[j]], dsem,
                             add=True)
            return 0
        lax.fori_loop(0, n_chunks, fire, 0)

        def drain(j, _):
            pltpu.make_async_copy(val_v.at[j], deg_sp.at[src_v.at[j]],
                                  dsem).wait()
            return 0
        lax.fori_loop(0, n_chunks, drain, 0)
        plsc.subcore_barrier()

        pltpu.sync_copy(rmp_v, rmp_hbm.at[t])
        pltpu.sync_copy(deg_sp.at[pl.ds(s * nz, nz)], buf_v)
        pltpu.sync_copy(buf_v, deg_hbm.at[c, pl.ds(s * nz, nz)])

    f = pl.kernel(
        body,
        out_type=(
            jax.ShapeDtypeStruct((NC, n_deg), jnp.float32),
            jax.ShapeDtypeStruct(src3.shape, jnp.int32),
        ),
        mesh=_sc_mesh(),
        scratch_types=(
            pltpu.VMEM((n_chunks, CHUNK), jnp.int32),
            pltpu.VMEM((n_chunks, CHUNK), jnp.int32),
            pltpu.VMEM((n_chunks, CHUNK), jnp.float32),
            pltpu.VMEM((n_chunks, CHUNK), jnp.int32),
            pltpu.VMEM((nz,), jnp.float32),
            pltpu.SemaphoreType.DMA,
            pltpu.VMEM_SHARED((n_deg,), jnp.float32),
        ),
    )
    return f(src3, dst3)


def _prescale_call(degt, h, bn):
    n, d = h.shape
    dh = d // 2

    def body(deg_ref, h_ref, hs_ref):
        dsum = deg_ref[:, 0:1] + deg_ref[:, 1:2]
        dis = jnp.where(dsum > 0, lax.rsqrt(dsum), 0.0)
        hs = h_ref[...] * dis
        hs_ref[0] = hs[:, :dh]
        hs_ref[1] = hs[:, dh:]

    return pl.pallas_call(
        body,
        grid=(n // bn,),
        in_specs=[
            pl.BlockSpec((bn, 2), lambda i: (i, 0)),
            pl.BlockSpec((bn, d), lambda i: (i, 0)),
        ],
        out_specs=pl.BlockSpec((2, bn, dh), lambda i: (0, i, 0)),
        out_shape=jax.ShapeDtypeStruct((2, n, dh), jnp.float32),
    )(degt, h)


CHUNK2 = 64


def _edge_scatter_call(hs, src2, rmp3, acc_rows):
    tile_e = src2.shape[1]
    n_chunks = tile_e // CHUNK2
    n_pairs = n_chunks // 2
    dh = hs.shape[2]
    nz = acc_rows // NS

    def body(hs_hbm, src_hbm, rmp_hbm, out_hbm,
             src_v, rmp_v, row_a, row_b, row_c,
             gsa, gsb, gsc, ssa, ssb, ssc, acc_sp):
        c = lax.axis_index("c")
        s = lax.axis_index("s")
        table = hs_hbm.at[c]
        pltpu.sync_copy(src_hbm.at[s], src_v)
        pltpu.sync_copy(rmp_hbm.at[s], rmp_v)

        def sidx(j):
            return rmp_v.at[pl.ds(j * CHUNK2, CHUNK2)]

        def zstep(r, _):
            for u in range(dh // LANE):
                row_a[r, pl.ds(u * LANE, LANE)] = jnp.zeros((LANE,),
                                                            jnp.float32)
            return 0
        lax.fori_loop(0, CHUNK2, zstep, 0)
        off = 0
        while off < nz:
            sz = min(CHUNK2, nz - off)
            pltpu.sync_copy(row_a.at[pl.ds(0, sz)],
                            acc_sp.at[pl.ds(s * nz + off, sz)])
            off += sz
        plsc.subcore_barrier()

        def gidx(j):
            return src_v.at[pl.ds(j * CHUNK2, CHUNK2)]

        pltpu.async_copy(table.at[gidx(0)], row_a, gsa)
        pltpu.async_copy(table.at[gidx(1)], row_b, gsb)
        ring = ((row_a, gsa, ssa), (row_b, gsb, ssb), (row_c, gsc, ssc))

        def step(q, _):
            j0 = 3 * q
            for k in range(3):
                j = j0 + k
                buf, gsem, ssem = ring[k]
                pbuf, pgsem, pssem = ring[(k + 2) % 3]
                pltpu.make_async_copy(table.at[gidx(j)], buf, gsem).wait()
                pltpu.async_copy(buf, acc_sp.at[sidx(j)], ssem, add=True)

                @pl.when(j >= 1)
                def _():
                    pltpu.make_async_copy(pbuf, acc_sp.at[sidx(j - 1)],
                                          pssem).wait()

                @pl.when(j + 2 < n_chunks)
                def _():
                    pltpu.async_copy(table.at[gidx(j + 2)], pbuf, pgsem)
            return 0
        lax.fori_loop(0, n_chunks // 3, step, 0)
        jt = n_chunks - 1
        tbuf, tgsem, tssem = ring[jt % 3]
        wbuf, wgsem, wssem = ring[(jt + 2) % 3]
        pltpu.make_async_copy(table.at[gidx(jt)], tbuf, tgsem).wait()
        pltpu.async_copy(tbuf, acc_sp.at[sidx(jt)], tssem, add=True)
        pltpu.make_async_copy(wbuf, acc_sp.at[sidx(jt - 1)], wssem).wait()
        pltpu.make_async_copy(tbuf, acc_sp.at[sidx(jt)], tssem).wait()
        plsc.subcore_barrier()

        off = 0
        while off < nz:
            sz = min(CHUNK2, nz - off)
            pltpu.sync_copy(acc_sp.at[pl.ds(s * nz + off, sz)],
                            row_a.at[pl.ds(0, sz)])
            pltpu.sync_copy(row_a.at[pl.ds(0, sz)],
                            out_hbm.at[c, pl.ds(s * nz + off, sz)])
            off += sz

    f = pl.kernel(
        body,
        out_type=jax.ShapeDtypeStruct((NC, acc_rows, dh), jnp.float32),
        mesh=_sc_mesh(),
        scratch_types=(
            pltpu.VMEM((tile_e,), jnp.int32),
            pltpu.VMEM((tile_e,), jnp.int32),
            pltpu.VMEM((CHUNK2, dh), jnp.float32),
            pltpu.VMEM((CHUNK2, dh), jnp.float32),
            pltpu.VMEM((CHUNK2, dh), jnp.float32),
            pltpu.SemaphoreType.DMA,
            pltpu.SemaphoreType.DMA,
            pltpu.SemaphoreType.DMA,
            pltpu.SemaphoreType.DMA,
            pltpu.SemaphoreType.DMA,
            pltpu.SemaphoreType.DMA,
            pltpu.VMEM_SHARED((acc_rows, dh), jnp.float32),
        ),
    )
    return f(hs, src2, rmp3)


def _dense_call(degt, nf, h, c, acc2, Wcat, T0cat, T1cat, bias, wc3,
                Wl, bl, bn):
    n, d = nf.shape
    d4 = 4 * d
    dh = d // 2

    def body(deg_ref, nf_ref, h_ref, c_ref, a_ref, wc_ref, t0_ref,
             t1_ref, bias_ref, wc3_ref, wl_ref, bl_ref,
             z_ref, hn_ref, cn_ref):
        dsum = deg_ref[:, 0:1] + deg_ref[:, 1:2]
        dis = jnp.where(dsum > 0, lax.rsqrt(dsum), 0.0)
        acc = jnp.concatenate([a_ref[0], a_ref[1]], axis=1)
        tx1 = acc * (-dis)
        P = jnp.dot(nf_ref[...], wc_ref[...],
                    preferred_element_type=jnp.float32)
        P = P + jnp.dot(h_ref[...], t0_ref[...],
                        preferred_element_type=jnp.float32)
        P = P + jnp.dot(tx1, t1_ref[...], preferred_element_type=jnp.float32)
        P = P + bias_ref[...]
        cb = c_ref[...]
        I = jax.nn.sigmoid(P[:, 0:d] + wc3_ref[0:1, :] * cb)
        F = jax.nn.sigmoid(P[:, d:2 * d] + wc3_ref[1:2, :] * cb)
        T = jnp.tanh(P[:, 2 * d:3 * d])
        Cn = F * cb + I * T
        O = jax.nn.sigmoid(P[:, 3 * d:4 * d] + wc3_ref[2:3, :] * Cn)
        Hn = O * jnp.tanh(Cn)
        z_ref[...] = jnp.dot(jnp.maximum(Hn, 0.0), wl_ref[...],
                             preferred_element_type=jnp.float32) + bl_ref[...]
        hn_ref[...] = Hn
        cn_ref[...] = Cn

    return pl.pallas_call(
        body,
        grid=(n // bn,),
        in_specs=[
            pl.BlockSpec((bn, 2), lambda i: (i, 0)),
            pl.BlockSpec((bn, d), lambda i: (i, 0)),
            pl.BlockSpec((bn, d), lambda i: (i, 0)),
            pl.BlockSpec((bn, d), lambda i: (i, 0)),
            pl.BlockSpec((2, bn, dh), lambda i: (0, i, 0)),
            pl.BlockSpec((d, d4), lambda i: (0, 0)),
            pl.BlockSpec((d, d4), lambda i: (0, 0)),
            pl.BlockSpec((d, d4), lambda i: (0, 0)),
            pl.BlockSpec((1, d4), lambda i: (0, 0)),
            pl.BlockSpec((3, d), lambda i: (0, 0)),
            pl.BlockSpec((d, d), lambda i: (0, 0)),
            pl.BlockSpec((1, d), lambda i: (0, 0)),
        ],
        out_specs=[pl.BlockSpec((bn, d), lambda i: (i, 0))] * 3,
        out_shape=[jax.ShapeDtypeStruct((n, d), jnp.float32)] * 3,
    )(degt, nf, h, c, acc2, Wcat, T0cat, T1cat, bias, wc3, Wl, bl)


def kernel(node_feat, src, dst, h, c, params):
    n, d = node_feat.shape
    e = src.shape[0]
    p = params

    e_pad = _round_up(e, NS * 4 * CHUNK2)
    pad = e_pad - e
    srcp = jnp.concatenate([src.astype(jnp.int32),
                            jnp.zeros((pad,), jnp.int32)])
    dstp = jnp.concatenate([dst.astype(jnp.int32),
                            jnp.zeros((pad,), jnp.int32)])
    src1 = srcp.reshape(NC * NS, -1, CHUNK)
    dst1 = dstp.reshape(NC * NS, -1, CHUNK)

    n_deg = _round_up(n, NS * LANE)
    acc_rows = _round_up(n + 1, NS * 8)
    trash = n

    deg_part, rmp1 = _deg_remap_call(src1, dst1, n_deg, trash)
    degt = deg_part.T

    bn = 1000
    hs = _prescale_call(degt, h, bn)

    src2 = srcp.reshape(NS, -1)
    rmp2 = rmp1.reshape(NS, -1)
    acc2 = _edge_scatter_call(hs, src2, rmp2, acc_rows)

    Wcat = jnp.concatenate([p["W_" + g] for g in "ifco"], axis=1)
    T0cat = jnp.concatenate([p["T0_" + g] for g in "ifco"], axis=1)
    T1cat = jnp.concatenate([p["T1_" + g] for g in "ifco"], axis=1)
    bias = jnp.concatenate([p["bc_" + g] + p["b_" + g][0]
                            for g in "ifco"])[None, :]
    wc3 = jnp.concatenate([p["wc_i"], p["wc_f"], p["wc_o"]], axis=0)
    bl = p["b_lin"][None, :]

    return _dense_call(degt, node_feat, h, c, acc2, Wcat, T0cat, T1cat,
                       bias, wc3, p["W_lin"], bl, bn)


# speedup vs baseline: 1.3504x; 1.3504x over previous
import jax
import jax.numpy as jnp
from jax import lax
from jax.experimental import pallas as pl
from jax.experimental.pallas import tpu as pltpu
from jax.experimental.pallas import tpu_sc as plsc

NC = 2
NS = 16
LANE = 16
CHUNK = 64


def _round_up(x, m):
    return (x + m - 1) // m * m


def _sc_mesh():
    return plsc.VectorSubcoreMesh(
        core_axis_name="c", subcore_axis_name="s", num_cores=NC,
        num_subcores=NS)


def _deg_remap_call(src3, dst3, n_deg, trash):
    n_chunks = src3.shape[1]
    nz = n_deg // NS

    def body(src_hbm, dst_hbm, deg_hbm, rmp_hbm,
             src_v, dst_v, val_v, rmp_v, buf_v, dsem, deg_sp):
        c = lax.axis_index("c")
        s = lax.axis_index("s")
        t = c * NS + s
        pltpu.sync_copy(src_hbm.at[t], src_v)
        pltpu.sync_copy(dst_hbm.at[t], dst_v)

        def zstep(i, _):
            buf_v[pl.ds(i * LANE, LANE)] = jnp.zeros((LANE,), jnp.float32)
            return 0
        lax.fori_loop(0, nz // LANE, zstep, 0)
        pltpu.sync_copy(buf_v, deg_sp.at[pl.ds(s * nz, nz)])
        plsc.subcore_barrier()

        def cstep(j, _):
            for u in range(CHUNK // LANE):
                sj = src_v[j, pl.ds(u * LANE, LANE)]
                dj = dst_v[j, pl.ds(u * LANE, LANE)]
                m = sj != dj
                val_v[j, pl.ds(u * LANE, LANE)] = jnp.where(m, 1.0, 0.0)
                rmp_v[j, pl.ds(u * LANE, LANE)] = jnp.where(m, dj, trash)
            return 0
        lax.fori_loop(0, n_chunks, cstep, 0)

        def fire(j, _):
            pltpu.async_copy(val_v.at[j], deg_sp.at[src_v.at[j]], dsem,
                             add=True)
            return 0
        lax.fori_loop(0, n_chunks, fire, 0)

        def drain(j, _):
            pltpu.make_async_copy(val_v.at[j], deg_sp.at[src_v.at[j]],
                                  dsem).wait()
            return 0
        lax.fori_loop(0, n_chunks, drain, 0)
        plsc.subcore_barrier()

        pltpu.sync_copy(rmp_v, rmp_hbm.at[t])
        pltpu.sync_copy(deg_sp.at[pl.ds(s * nz, nz)], buf_v)
        pltpu.sync_copy(buf_v, deg_hbm.at[c, pl.ds(s * nz, nz)])

    f = pl.kernel(
        body,
        out_type=(
            jax.ShapeDtypeStruct((NC, n_deg), jnp.float32),
            jax.ShapeDtypeStruct(src3.shape, jnp.int32),
        ),
        mesh=_sc_mesh(),
        scratch_types=(
            pltpu.VMEM((n_chunks, CHUNK), jnp.int32),
            pltpu.VMEM((n_chunks, CHUNK), jnp.int32),
            pltpu.VMEM((n_chunks, CHUNK), jnp.float32),
            pltpu.VMEM((n_chunks, CHUNK), jnp.int32),
            pltpu.VMEM((nz,), jnp.float32),
            pltpu.SemaphoreType.DMA,
            pltpu.VMEM_SHARED((n_deg,), jnp.float32),
        ),
    )
    return f(src3, dst3)


def _prescale_call(degt, h, bn):
    n, d = h.shape
    dh = d // 2

    def body(deg_ref, h_ref, hs_ref):
        dsum = deg_ref[:, 0:1] + deg_ref[:, 1:2]
        dis = jnp.where(dsum > 0, lax.rsqrt(dsum), 0.0)
        hs = h_ref[...] * dis
        hs_ref[0] = hs[:, :dh]
        hs_ref[1] = hs[:, dh:]

    return pl.pallas_call(
        body,
        grid=(n // bn,),
        in_specs=[
            pl.BlockSpec((bn, 2), lambda i: (i, 0)),
            pl.BlockSpec((bn, d), lambda i: (i, 0)),
        ],
        out_specs=pl.BlockSpec((2, bn, dh), lambda i: (0, i, 0)),
        out_shape=jax.ShapeDtypeStruct((2, n, dh), jnp.float32),
    )(degt, h)


CHUNK2 = 32


def _edge_scatter_call(hs, src2, rmp3, acc_rows):
    tile_e = src2.shape[1]
    n_chunks = tile_e // CHUNK2
    n_pairs = n_chunks // 2
    dh = hs.shape[2]
    nz = acc_rows // NS

    def body(hs_hbm, src_hbm, rmp_hbm, out_hbm,
             src_v, rmp_v, row_a, row_b, row_c, row_d,
             gsa, gsb, gsc, gsd, ssa, ssb, ssc, ssd, acc_sp):
        c = lax.axis_index("c")
        s = lax.axis_index("s")
        table = hs_hbm.at[c]
        pltpu.sync_copy(src_hbm.at[s], src_v)
        pltpu.sync_copy(rmp_hbm.at[s], rmp_v)

        def sidx(j):
            return rmp_v.at[pl.ds(j * CHUNK2, CHUNK2)]

        def zstep(r, _):
            for u in range(dh // LANE):
                row_a[r, pl.ds(u * LANE, LANE)] = jnp.zeros((LANE,),
                                                            jnp.float32)
            return 0
        lax.fori_loop(0, CHUNK2, zstep, 0)
        off = 0
        while off < nz:
            sz = min(CHUNK2, nz - off)
            pltpu.sync_copy(row_a.at[pl.ds(0, sz)],
                            acc_sp.at[pl.ds(s * nz + off, sz)])
            off += sz
        plsc.subcore_barrier()

        def gidx(j):
            return src_v.at[pl.ds(j * CHUNK2, CHUNK2)]

        pltpu.async_copy(table.at[gidx(0)], row_a, gsa)
        pltpu.async_copy(table.at[gidx(1)], row_b, gsb)
        pltpu.async_copy(table.at[gidx(2)], row_c, gsc)
        ring = ((row_a, gsa, ssa), (row_b, gsb, ssb),
                (row_c, gsc, ssc), (row_d, gsd, ssd))

        def step(q, _):
            j0 = 4 * q
            for k in range(4):
                j = j0 + k
                buf, gsem, ssem = ring[k]
                pbuf, pgsem, pssem = ring[(k + 3) % 4]
                pltpu.make_async_copy(table.at[gidx(j)], buf, gsem).wait()
                pltpu.async_copy(buf, acc_sp.at[sidx(j)], ssem, add=True)

                @pl.when(j >= 1)
                def _():
                    pltpu.make_async_copy(pbuf, acc_sp.at[sidx(j - 1)],
                                          pssem).wait()

                @pl.when(j + 3 < n_chunks)
                def _():
                    pltpu.async_copy(table.at[gidx(j + 3)], pbuf, pgsem)
            return 0
        lax.fori_loop(0, n_chunks // 4, step, 0)
        pltpu.make_async_copy(row_d, acc_sp.at[sidx(n_chunks - 1)],
                              ssd).wait()
        plsc.subcore_barrier()

        off = 0
        while off < nz:
            sz = min(CHUNK2, nz - off)
            pltpu.sync_copy(acc_sp.at[pl.ds(s * nz + off, sz)],
                            row_a.at[pl.ds(0, sz)])
            pltpu.sync_copy(row_a.at[pl.ds(0, sz)],
                            out_hbm.at[c, pl.ds(s * nz + off, sz)])
            off += sz

    f = pl.kernel(
        body,
        out_type=jax.ShapeDtypeStruct((NC, acc_rows, dh), jnp.float32),
        mesh=_sc_mesh(),
        scratch_types=(
            pltpu.VMEM((tile_e,), jnp.int32),
            pltpu.VMEM((tile_e,), jnp.int32),
            pltpu.VMEM((CHUNK2, dh), jnp.float32),
            pltpu.VMEM((CHUNK2, dh), jnp.float32),
            pltpu.VMEM((CHUNK2, dh), jnp.float32),
            pltpu.VMEM((CHUNK2, dh), jnp.float32),
            pltpu.SemaphoreType.DMA,
            pltpu.SemaphoreType.DMA,
            pltpu.SemaphoreType.DMA,
            pltpu.SemaphoreType.DMA,
            pltpu.SemaphoreType.DMA,
            pltpu.SemaphoreType.DMA,
            pltpu.SemaphoreType.DMA,
            pltpu.SemaphoreType.DMA,
            pltpu.VMEM_SHARED((acc_rows, dh), jnp.float32),
        ),
    )
    return f(hs, src2, rmp3)


def _dense_call(degt, nf, h, c, acc2, Wcat, T0cat, T1cat, bias, wc3,
                Wl, bl, bn):
    n, d = nf.shape
    d4 = 4 * d
    dh = d // 2

    def body(deg_ref, nf_ref, h_ref, c_ref, a_ref, wc_ref, t0_ref,
             t1_ref, bias_ref, wc3_ref, wl_ref, bl_ref,
             z_ref, hn_ref, cn_ref):
        dsum = deg_ref[:, 0:1] + deg_ref[:, 1:2]
        dis = jnp.where(dsum > 0, lax.rsqrt(dsum), 0.0)
        acc = jnp.concatenate([a_ref[0], a_ref[1]], axis=1)
        tx1 = acc * (-dis)
        P = jnp.dot(nf_ref[...], wc_ref[...],
                    preferred_element_type=jnp.float32)
        P = P + jnp.dot(h_ref[...], t0_ref[...],
                        preferred_element_type=jnp.float32)
        P = P + jnp.dot(tx1, t1_ref[...], preferred_element_type=jnp.float32)
        P = P + bias_ref[...]
        cb = c_ref[...]
        I = jax.nn.sigmoid(P[:, 0:d] + wc3_ref[0:1, :] * cb)
        F = jax.nn.sigmoid(P[:, d:2 * d] + wc3_ref[1:2, :] * cb)
        T = jnp.tanh(P[:, 2 * d:3 * d])
        Cn = F * cb + I * T
        O = jax.nn.sigmoid(P[:, 3 * d:4 * d] + wc3_ref[2:3, :] * Cn)
        Hn = O * jnp.tanh(Cn)
        z_ref[...] = jnp.dot(jnp.maximum(Hn, 0.0), wl_ref[...],
                             preferred_element_type=jnp.float32) + bl_ref[...]
        hn_ref[...] = Hn
        cn_ref[...] = Cn

    return pl.pallas_call(
        body,
        grid=(n // bn,),
        in_specs=[
            pl.BlockSpec((bn, 2), lambda i: (i, 0)),
            pl.BlockSpec((bn, d), lambda i: (i, 0)),
            pl.BlockSpec((bn, d), lambda i: (i, 0)),
            pl.BlockSpec((bn, d), lambda i: (i, 0)),
            pl.BlockSpec((2, bn, dh), lambda i: (0, i, 0)),
            pl.BlockSpec((d, d4), lambda i: (0, 0)),
            pl.BlockSpec((d, d4), lambda i: (0, 0)),
            pl.BlockSpec((d, d4), lambda i: (0, 0)),
            pl.BlockSpec((1, d4), lambda i: (0, 0)),
            pl.BlockSpec((3, d), lambda i: (0, 0)),
            pl.BlockSpec((d, d), lambda i: (0, 0)),
            pl.BlockSpec((1, d), lambda i: (0, 0)),
        ],
        out_specs=[pl.BlockSpec((bn, d), lambda i: (i, 0))] * 3,
        out_shape=[jax.ShapeDtypeStruct((n, d), jnp.float32)] * 3,
    )(degt, nf, h, c, acc2, Wcat, T0cat, T1cat, bias, wc3, Wl, bl)


def kernel(node_feat, src, dst, h, c, params):
    n, d = node_feat.shape
    e = src.shape[0]
    p = params

    e_pad = _round_up(e, NC * NS * CHUNK)
    pad = e_pad - e
    srcp = jnp.concatenate([src.astype(jnp.int32),
                            jnp.zeros((pad,), jnp.int32)])
    dstp = jnp.concatenate([dst.astype(jnp.int32),
                            jnp.zeros((pad,), jnp.int32)])
    src1 = srcp.reshape(NC * NS, -1, CHUNK)
    dst1 = dstp.reshape(NC * NS, -1, CHUNK)

    n_deg = _round_up(n, NS * LANE)
    acc_rows = _round_up(n + 1, NS * 8)
    trash = n

    deg_part, rmp1 = _deg_remap_call(src1, dst1, n_deg, trash)
    degt = deg_part.T

    bn = 1000
    hs = _prescale_call(degt, h, bn)

    src2 = srcp.reshape(NS, -1)
    rmp2 = rmp1.reshape(NS, -1)
    acc2 = _edge_scatter_call(hs, src2, rmp2, acc_rows)

    Wcat = jnp.concatenate([p["W_" + g] for g in "ifco"], axis=1)
    T0cat = jnp.concatenate([p["T0_" + g] for g in "ifco"], axis=1)
    T1cat = jnp.concatenate([p["T1_" + g] for g in "ifco"], axis=1)
    bias = jnp.concatenate([p["bc_" + g] + p["b_" + g][0]
                            for g in "ifco"])[None, :]
    wc3 = jnp.concatenate([p["wc_i"], p["wc_f"], p["wc_o"]], axis=0)
    bl = p["b_lin"][None, :]

    return _dense_call(degt, node_feat, h, c, acc2, Wcat, T0cat, T1cat,
                       bias, wc3, p["W_lin"], bl, bn)
